# baseline (speedup 1.0000x reference)
import functools

import numpy as np
import jax
import jax.numpy as jnp
from jax import lax
from jax.experimental import pallas as pl
from jax.experimental.pallas import tpu as pltpu
from jax.experimental.pallas import tpu_sc as plsc

_D = 64
_SEQ = 2048
_BATCH = 16

_NC, _NS = 2, 16
_NW = _NC * _NS
_B = _SEQ * _BATCH
_BPW = _B // _NW
_NCHUNK = 8
_CH = _BPW // _NCHUNK
_GPW = _BPW // _BATCH


def _pe_table():
    position = np.arange(0, _SEQ, dtype=np.float32)[:, None]
    div_term = np.exp(
        np.arange(0, _D, 2).astype(np.float32) * (-np.log(10000.0) / _D)
    )
    pe = np.zeros((_SEQ, _D), dtype=np.float32)
    pe[:, 0::2] = np.sin(position * div_term)
    pe[:, 1::2] = np.cos(position * div_term)
    return pe


_PE = _pe_table()


def _sc_embed(W, xf, pe):
    mesh = plsc.VectorSubcoreMesh(core_axis_name="c", subcore_axis_name="s")

    @functools.partial(
        pl.kernel,
        mesh=mesh,
        out_type=jax.ShapeDtypeStruct((_B, _D), jnp.float32),
        scratch_types=[
            pltpu.VMEM((_NCHUNK, _CH), jnp.int32),
            pltpu.VMEM((_BPW, _D), jnp.float32),
            pltpu.VMEM((_GPW, _D), jnp.float32),
            pltpu.SemaphoreType.DMA,
        ],
        compiler_params=pltpu.CompilerParams(use_tc_tiling_on_sc=False),
    )
    def k(w_hbm, x_hbm, pe_hbm, out_hbm, idx_v, rows_v, pe_v, sem):
        wid = lax.axis_index("s") * _NC + lax.axis_index("c")
        base = wid * _BPW
        pltpu.sync_copy(x_hbm.at[wid], idx_v)
        copies = [
            pltpu.async_copy(
                w_hbm.at[idx_v.at[j]], rows_v.at[pl.ds(j * _CH, _CH)], sem
            )
            for j in range(_NCHUNK)
        ]
        pltpu.sync_copy(pe_hbm.at[pl.ds(wid * _GPW, _GPW)], pe_v)
        for c in copies:
            c.wait()

        def body(g, carry):
            for c in range(_D // 16):
                pev = pe_v[g, pl.ds(c * 16, 16)]
                for r in range(_BATCH):
                    row = g * _BATCH + r
                    rows_v[row, pl.ds(c * 16, 16)] = (
                        rows_v[row, pl.ds(c * 16, 16)] + pev
                    )
            return carry

        lax.fori_loop(0, _GPW, body, 0)
        pltpu.sync_copy(rows_v, out_hbm.at[pl.ds(base, _BPW)])

    return k(W, xf, pe)


def kernel(x, W):
    xf = x.reshape(_NW, _NCHUNK, _CH)
    pe = jnp.asarray(_PE)
    out = _sc_embed(W, xf, pe)
    return out.reshape(_SEQ, _BATCH, _D)
